# baseline (speedup 1.0000x reference)
import functools

import jax
import jax.numpy as jnp
import numpy as np
from jax import lax
from jax.experimental import pallas as pl
from jax.experimental.pallas import tpu as pltpu
from jax.experimental.pallas import tpu_sc as plsc

NC = 2
NS = 16
NW = NC * NS
B = 4096
L = 50
D = 64
EPW = B // NW
EPO = 2
OPW = EPW // EPO
ND = D // 16
NBUF = 2

_mesh = plsc.VectorSubcoreMesh(
    core_axis_name="c", subcore_axis_name="s", num_cores=NC, num_subcores=NS)


@functools.partial(
    pl.kernel,
    out_type=jax.ShapeDtypeStruct((B, D), jnp.float32),
    mesh=_mesh,
    scratch_types=[
        pltpu.VMEM((OPW, EPO * L), jnp.int32),
        [pltpu.VMEM((EPO * L, D), jnp.bfloat16) for _ in range(NBUF)],
        pltpu.VMEM((EPW, D), jnp.float32),
        pltpu.VMEM((EPW, 16), jnp.int32),
        [pltpu.SemaphoreType.DMA for _ in range(NBUF)],
    ],
    compiler_params=pltpu.CompilerParams(use_tc_tiling_on_sc=False, needs_layout_passes=False),
)
def _encode(tok2, lens, table, out, idx_v, bufs, outb, lens_v, sems):
    wid = lax.axis_index("c") * NS + lax.axis_index("s")
    base = wid * EPW
    pltpu.sync_copy(tok2.at[pl.ds(wid * OPW, OPW)], idx_v)
    pltpu.sync_copy(lens.at[pl.ds(base, EPW)], lens_v)

    def start(j, b):
        pltpu.async_copy(table.at[idx_v.at[j]], bufs[b], sems[b])

    for b in range(NBUF):
        start(b, b)

    @pl.loop(0, OPW, step=NBUF)
    def _per_group(j0):
        for b in range(NBUF):
            j = j0 + b
            pltpu.make_async_copy(
                table.at[idx_v.at[j]], bufs[b], sems[b]).wait()
            buf = bufs[b]
            for e in range(EPO):
                eloc = EPO * j + e
                accs = [None] * ND
                for r in range(L):
                    for h in range(2):
                        v = buf[L * e + r, pl.ds(h * 32, 32)]
                        va, vb = plsc.unpack(
                            v, format=plsc.PackFormat.INTERLEAVED)
                        if r == 0:
                            accs[2 * h] = va
                            accs[2 * h + 1] = vb
                        else:
                            accs[2 * h] = accs[2 * h] + va
                            accs[2 * h + 1] = accs[2 * h + 1] + vb
                lvec = lens_v[eloc, pl.ds(0, 16)]
                inv = 1.0 / lvec.astype(jnp.float32)
                for d in range(ND):
                    outb[eloc, pl.ds(d * 16, 16)] = accs[d] * inv

            @pl.when(j + NBUF < OPW)
            def _refill():
                start(j + NBUF, b)

    pltpu.sync_copy(outb, out.at[pl.ds(base, EPW)])


_PERM = np.concatenate(
    [np.stack([np.arange(16) + b, np.arange(16) + b + 16], axis=1).ravel()
     for b in (0, 32)])


def kernel(tok_batch, tok_lens, table):
    table = table.astype(jnp.bfloat16)[:, _PERM]
    tok2 = tok_batch.reshape(B // EPO, EPO * L)
    lens16 = jnp.broadcast_to(tok_lens[:, None], (B, 16))
    return _encode(tok2, lens16, table)


# speedup vs baseline: 1.1870x; 1.1870x over previous
import functools

import jax
import jax.numpy as jnp
import numpy as np
from jax import lax
from jax.experimental import pallas as pl
from jax.experimental.pallas import tpu as pltpu
from jax.experimental.pallas import tpu_sc as plsc

NC = 2
NS = 16
NW = NC * NS
B = 4096
L = 50
D = 64
EPW = B // NW
EPO = 2
OPW = EPW // EPO
ND = D // 16
NBUF = 2

_mesh = plsc.VectorSubcoreMesh(
    core_axis_name="c", subcore_axis_name="s", num_cores=NC, num_subcores=NS)


@functools.partial(
    pl.kernel,
    out_type=jax.ShapeDtypeStruct((B, D), jnp.float32),
    mesh=_mesh,
    scratch_types=[
        pltpu.VMEM((OPW, EPO * L), jnp.int32),
        [pltpu.VMEM((EPO * L, D), jnp.bfloat16) for _ in range(NBUF)],
        pltpu.VMEM((EPW, D), jnp.float32),
        pltpu.VMEM((EPW, 16), jnp.int32),
        [pltpu.SemaphoreType.DMA for _ in range(NBUF)],
    ],
    compiler_params=pltpu.CompilerParams(use_tc_tiling_on_sc=False, needs_layout_passes=False),
)
def _encode(tok2, lens, table, out, idx_v, bufs, outb, lens_v, sems):
    wid = lax.axis_index("c") * NS + lax.axis_index("s")
    base = wid * EPW
    pltpu.sync_copy(tok2.at[pl.ds(wid * OPW, OPW)], idx_v)
    pltpu.sync_copy(lens.at[pl.ds(base, EPW)], lens_v)

    def start(j, b):
        pltpu.async_copy(table.at[idx_v.at[j]], bufs[b], sems[b])

    for b in range(NBUF):
        start(b, b)

    @pl.loop(0, OPW, step=NBUF)
    def _per_group(j0):
        for b in range(NBUF):
            j = j0 + b
            pltpu.make_async_copy(
                table.at[idx_v.at[j]], bufs[b], sems[b]).wait()
            buf = bufs[b]
            for e in range(EPO):
                eloc = EPO * j + e
                accs = [None] * ND
                for r in range(L):
                    for h in range(2):
                        v = buf[L * e + r, pl.ds(h * 32, 32)]
                        va, vb = plsc.unpack(
                            v, format=plsc.PackFormat.INTERLEAVED)
                        if r == 0:
                            accs[2 * h] = va
                            accs[2 * h + 1] = vb
                        else:
                            accs[2 * h] = accs[2 * h] + va
                            accs[2 * h + 1] = accs[2 * h + 1] + vb
                lvec = lens_v[eloc, pl.ds(0, 16)]
                inv = 1.0 / lvec.astype(jnp.float32)
                for d in range(ND):
                    outb[eloc, pl.ds(d * 16, 16)] = accs[d] * inv

            @pl.when(j + NBUF < OPW)
            def _refill():
                start(j + NBUF, b)

    pltpu.sync_copy(outb, out.at[pl.ds(base, EPW)])


_PERM = np.concatenate(
    [np.stack([np.arange(16) + b, np.arange(16) + b + 16], axis=1).ravel()
     for b in (0, 32)])


def kernel(tok_batch, tok_lens, table):
    table = table.astype(jnp.bfloat16)
    tok2 = tok_batch.reshape(B // EPO, EPO * L)
    lens16 = jnp.broadcast_to(tok_lens[:, None], (B, 16))
    return _encode(tok2, lens16, table)


# speedup vs baseline: 1.3808x; 1.1633x over previous
import functools

import jax
import jax.numpy as jnp
from jax import lax
from jax.experimental import pallas as pl
from jax.experimental.pallas import tpu as pltpu
from jax.experimental.pallas import tpu_sc as plsc

NC = 2
NS = 16
NW = NC * NS
B = 4096
L = 50
D = 64
EPW = B // NW
EPO = 2
OPW = EPW // EPO
ND = D // 16
NBUF = 2

_mesh = plsc.VectorSubcoreMesh(
    core_axis_name="c", subcore_axis_name="s", num_cores=NC, num_subcores=NS)


@functools.partial(
    pl.kernel,
    out_type=jax.ShapeDtypeStruct((B, D), jnp.float32),
    mesh=_mesh,
    scratch_types=[
        pltpu.VMEM((OPW, EPO * L), jnp.int32),
        [pltpu.VMEM((EPO * L, D), jnp.float32) for _ in range(NBUF)],
        pltpu.VMEM((EPW, D), jnp.float32),
        pltpu.VMEM((EPW,), jnp.int32),
        [pltpu.SemaphoreType.DMA for _ in range(NBUF)],
    ],
    compiler_params=pltpu.CompilerParams(use_tc_tiling_on_sc=False),
)
def _encode(tok2, lens, table, out, idx_v, bufs, outb, lens_v, sems):
    wid = lax.axis_index("c") * NS + lax.axis_index("s")
    base = wid * EPW
    pltpu.sync_copy(tok2.at[pl.ds(wid * OPW, OPW)], idx_v)
    pltpu.sync_copy(lens.at[pl.ds(base, EPW)], lens_v)

    def start(j, b):
        pltpu.async_copy(table.at[idx_v.at[j]], bufs[b], sems[b])

    for b in range(NBUF):
        start(b, b)

    @pl.loop(0, OPW, step=NBUF)
    def _per_group(j0):
        for b in range(NBUF):
            j = j0 + b
            pltpu.make_async_copy(
                table.at[idx_v.at[j]], bufs[b], sems[b]).wait()
            buf = bufs[b]
            for e in range(EPO):
                eloc = EPO * j + e
                accs = [buf[L * e, pl.ds(d * 16, 16)] for d in range(ND)]
                for r in range(1, L):
                    for d in range(ND):
                        accs[d] = accs[d] + buf[L * e + r, pl.ds(d * 16, 16)]
                chunk = lens_v[pl.ds((eloc // 16) * 16, 16)]
                lvec = chunk[jnp.full((16,), eloc % 16, jnp.int32)]
                inv = 1.0 / lvec.astype(jnp.float32)
                for d in range(ND):
                    outb[eloc, pl.ds(d * 16, 16)] = accs[d] * inv

            @pl.when(j + NBUF < OPW)
            def _refill():
                start(j + NBUF, b)

    pltpu.sync_copy(outb, out.at[pl.ds(base, EPW)])


def kernel(tok_batch, tok_lens, table):
    tok2 = tok_batch.reshape(B // EPO, EPO * L)
    return _encode(tok2, tok_lens, table)
